# baseline (speedup 1.0000x reference)
import functools

import numpy as np
import jax
import jax.numpy as jnp
from jax import lax
from jax.experimental import pallas as pl
from jax.experimental.pallas import tpu as pltpu
from jax.experimental.pallas import tpu_sc as plsc

_B = 16384
_NF = 26
_D = 16
_DIN = _NF * _D
_NPAIRS = _NF * (_NF - 1) // 2
_V = 2600000

_NC, _NS = 2, 16
_NW = _NC * _NS
_BPW = _B // _NW
_CH = 128
_NCH = _BPW // _CH
_ROWS = _CH * _NF

_CW = 1536
_NCHUNK = _V // _CW
_NITER6 = 54
_TAILC = _V - _NCHUNK * _CW

_bB = 1024
_NB = _B // _bB

_OFFSETS = (100000 * np.arange(_NF, dtype=np.int32))
_I1, _I2 = np.triu_indices(_NF, k=1)


def _tr_body(w2t_h, tail_h, out_h, ab0, ab1, ab2, o0, o1,
             si0, si1, si2, so0, so1):
    wid = lax.axis_index("s") * _NC + lax.axis_index("c")
    abs_ = (ab0, ab1, ab2)
    os_ = (o0, o1)
    sis = (si0, si1, si2)
    sos = (so0, so1)

    def start_in(cid, b):
        @pl.when(cid < _NCHUNK)
        def _():
            c0 = pl.multiple_of(cid * _CW, 128)
            pltpu.async_copy(w2t_h.at[:, pl.ds(c0, _CW)], abs_[b], sis[b])

    def wait_in(cid, b):
        @pl.when(cid < _NCHUNK)
        def _():
            pltpu.make_async_copy(w2t_h.at[:, pl.ds(0, _CW)],
                                  abs_[b], sis[b]).wait()

    def wait_out(cid, b):
        @pl.when(jnp.logical_and(cid >= 0, cid < _NCHUNK))
        def _():
            pltpu.make_async_copy(os_[b], out_h.at[pl.ds(0, _CW * _D)],
                                  sos[b]).wait()

    def scatter(bi, bo):
        ab_v, o_v = abs_[bi], os_[bo]

        def scat(g, carry):
            o = g * 16
            rows = (lax.iota(jnp.int32, 16) + o) * _D
            for d in range(_D):
                v = ab_v[d, pl.ds(o, 16)]
                plsc.store_scatter(o_v, [rows + d], v)
            return carry

        lax.fori_loop(0, _CW // 16, scat, 0)

    def half(cid, bi, bo):
        wait_in(cid, bi)
        start_in(cid + 2 * _NW, (bi + 2) % 3)
        wait_out(cid - 2 * _NW, bo)

        @pl.when(cid < _NCHUNK)
        def _():
            scatter(bi, bo)
            c0 = pl.multiple_of(cid * _CW, 128)
            pltpu.async_copy(os_[bo], out_h.at[pl.ds(c0 * _D, _CW * _D)],
                             sos[bo])

    start_in(wid, 0)
    start_in(wid + _NW, 1)

    def body6(t, carry):
        i0 = 6 * t
        for k in range(6):
            half(wid + (i0 + k) * _NW, k % 3, k % 2)
        return carry

    lax.fori_loop(0, _NITER6 // 6, body6, 0)
    wait_out(wid + (_NITER6 - 2) * _NW, 0)
    wait_out(wid + (_NITER6 - 1) * _NW, 1)

    @pl.when(wid == 0)
    def _():
        pltpu.sync_copy(tail_h, o0.at[pl.ds(0, _TAILC * _D)])
        pltpu.sync_copy(o0.at[pl.ds(0, _TAILC * _D)],
                        out_h.at[pl.ds(_NCHUNK * _CW * _D, _TAILC * _D)])


def _sc_transpose(w2t, tail1d):
    mesh = plsc.VectorSubcoreMesh(core_axis_name="c", subcore_axis_name="s")
    kfn = functools.partial(
        pl.kernel, mesh=mesh,
        compiler_params=pltpu.CompilerParams(use_tc_tiling_on_sc=True,
                                             needs_layout_passes=False),
        out_type=jax.ShapeDtypeStruct((_V * _D,), jnp.float32),
        scratch_types=[
            pltpu.VMEM((_D, _CW), jnp.float32),
            pltpu.VMEM((_D, _CW), jnp.float32),
            pltpu.VMEM((_D, _CW), jnp.float32),
            pltpu.VMEM((_CW * _D,), jnp.float32),
            pltpu.VMEM((_CW * _D,), jnp.float32),
            pltpu.SemaphoreType.DMA,
            pltpu.SemaphoreType.DMA,
            pltpu.SemaphoreType.DMA,
            pltpu.SemaphoreType.DMA,
            pltpu.SemaphoreType.DMA,
        ],
    )(_tr_body)
    return kfn(w2t, tail1d)


def _sc_body(W2_h, idx2_h, W1_h, idx1_h, h_out, e1_out,
             idx_v, rows_v, idx1_v, vals_v, acc_v, sem):
    wid = lax.axis_index("s") * _NC + lax.axis_index("c")
    base = wid * _BPW
    for c in range(_NCH):
        i0 = (base + c * _CH) * _NF
        pltpu.sync_copy(idx2_h.at[pl.ds(i0, _ROWS)], idx_v)
        pltpu.async_copy(W2_h.at[idx_v], rows_v, sem).wait()
        pltpu.sync_copy(rows_v, h_out.at[pl.ds(i0, _ROWS)])
    pltpu.sync_copy(idx1_h.at[wid], idx1_v)
    pltpu.async_copy(W1_h.at[idx1_v], vals_v, sem).wait()

    def red(c2, carry):
        o = c2 * 16
        s = vals_v[pl.ds(o, 16)]
        for f in range(1, _NF):
            s = s + vals_v[pl.ds(f * _BPW + o, 16)]
        acc_v[pl.ds(o, 16)] = s
        return carry

    lax.fori_loop(0, _BPW // 16, red, 0)
    pltpu.sync_copy(acc_v, e1_out.at[pl.ds(base, _BPW)])


def _sc_gather(W2row, idx2, W1v, idx1):
    mesh = plsc.VectorSubcoreMesh(core_axis_name="c", subcore_axis_name="s")
    kfn = functools.partial(
        pl.kernel, mesh=mesh,
        compiler_params=pltpu.CompilerParams(use_tc_tiling_on_sc=False),
        out_type=[jax.ShapeDtypeStruct((_B * _NF, _D), jnp.float32),
                  jax.ShapeDtypeStruct((_B,), jnp.float32)],
        scratch_types=[
            pltpu.VMEM((_ROWS,), jnp.int32),
            pltpu.VMEM((_ROWS, _D), jnp.float32),
            pltpu.VMEM((_NF * _BPW,), jnp.int32),
            pltpu.VMEM((_NF * _BPW,), jnp.float32),
            pltpu.VMEM((_BPW,), jnp.float32),
            pltpu.SemaphoreType.DMA,
        ],
    )(_sc_body)
    return kfn(W2row, idx2, W1v, idx1)


def _tc_body(h_ref, e1_ref, w1_ref, wb_ref, l2_ref, g1_ref, b1_ref,
             g2_ref, b2_ref, m3_ref, scl_ref, out_ref,
             pre1_s, pre2_s, cross_s, st1, st2):
    p = pl.program_id(0)
    j = pl.program_id(1)

    @pl.when(p == 0)
    def _p0():
        h = h_ref[...]
        pre1 = jnp.dot(h, w1_ref[...], preferred_element_type=jnp.float32)
        g = jnp.dot(h, wb_ref[...], preferred_element_type=jnp.float32)
        cross_s[pl.ds(j * _bB, _bB), :] = 0.5 * jnp.sum(h * g, axis=1,
                                                        keepdims=True)
        pre1_s[pl.ds(j * _bB, _bB), :] = pre1

        @pl.when(j == 0)
        def _z1():
            st1[...] = jnp.zeros((2, 128), jnp.float32)

        st1[...] += jnp.concatenate(
            [jnp.sum(pre1, 0, keepdims=True),
             jnp.sum(pre1 * pre1, 0, keepdims=True)], axis=0)

    @pl.when(p == 1)
    def _p1():
        mu = st1[0:1, :] * (1.0 / _B)
        var = st1[1:2, :] * (1.0 / _B) - mu * mu
        sc = g1_ref[...] * lax.rsqrt(var + 1e-5)
        pre1 = pre1_s[pl.ds(j * _bB, _bB), :]
        h1 = jnp.maximum((pre1 - mu) * sc + b1_ref[...], 0.0)
        pre2 = jnp.dot(h1, l2_ref[...], preferred_element_type=jnp.float32)
        pre2_s[pl.ds(j * _bB, _bB), :] = pre2

        @pl.when(j == 0)
        def _z2():
            st2[...] = jnp.zeros((2, 64), jnp.float32)

        st2[...] += jnp.concatenate(
            [jnp.sum(pre2, 0, keepdims=True),
             jnp.sum(pre2 * pre2, 0, keepdims=True)], axis=0)

    @pl.when(p == 2)
    def _p2():
        mu2 = st2[0:1, :] * (1.0 / _B)
        var2 = st2[1:2, :] * (1.0 / _B) - mu2 * mu2
        sc2 = g2_ref[...] * lax.rsqrt(var2 + 1e-5)
        pre2 = pre2_s[pl.ds(j * _bB, _bB), :]
        h2 = jnp.maximum((pre2 - mu2) * sc2 + b2_ref[...], 0.0)
        deep = jnp.sum(h2 * m3_ref[...], axis=1, keepdims=True)
        logit = (deep + e1_ref[...] * scl_ref[0:1, 0:1]
                 + cross_s[pl.ds(j * _bB, _bB), :] + scl_ref[0:1, 1:2])
        out_ref[...] = (1.0 / (1.0 + jnp.exp(-logit)))[None]


def _tc_dense(h2d, e1c, w1mat, Wbig, l2wT, g1, b1, g2, b2, m3row, scl):
    z2 = lambda p, j: (0, 0)
    return pl.pallas_call(
        _tc_body,
        grid=(3, _NB),
        in_specs=[
            pl.BlockSpec((_bB, _DIN), lambda p, j: (jnp.where(p == 0, j, 0), 0)),
            pl.BlockSpec((_bB, 1), lambda p, j: (jnp.where(p == 2, j, 0), 0)),
            pl.BlockSpec((_DIN, 128), z2),
            pl.BlockSpec((_DIN, _DIN), z2),
            pl.BlockSpec((128, 64), z2),
            pl.BlockSpec((1, 128), z2),
            pl.BlockSpec((1, 128), z2),
            pl.BlockSpec((1, 64), z2),
            pl.BlockSpec((1, 64), z2),
            pl.BlockSpec((1, 64), z2),
            pl.BlockSpec((1, 2), z2),
        ],
        out_specs=pl.BlockSpec((1, _bB, 1), lambda p, j: (p, j, 0)),
        out_shape=jax.ShapeDtypeStruct((3, _B, 1), jnp.float32),
        scratch_shapes=[
            pltpu.VMEM((_B, 128), jnp.float32),
            pltpu.VMEM((_B, 64), jnp.float32),
            pltpu.VMEM((_B, 1), jnp.float32),
            pltpu.VMEM((2, 128), jnp.float32),
            pltpu.VMEM((2, 64), jnp.float32),
        ],
    )(h2d, e1c, w1mat, Wbig, l2wT, g1, b1, g2, b2, m3row, scl)


def kernel(x, W1, W2, l1w, l1b, bn1g, bn1b, l2w, l2b, bn2g, bn2b,
           l3w, l3b, outw, outb):
    offs = jnp.asarray(_OFFSETS)
    xo = x + offs[None, :]
    idx2 = xo.reshape(_B * _NF)
    idx1 = xo.reshape(_NW, _BPW, _NF).transpose(0, 2, 1).reshape(_NW, _NF * _BPW)
    W1v = W1.reshape(-1)

    tail1d = W2[_NCHUNK * _CW:, :].reshape(_TAILC * _D)
    W2lin = _sc_transpose(W2.T, tail1d)
    W2row = W2lin.reshape(_V, _D)

    h_flat, e1 = _sc_gather(W2row, idx2, W1v, idx1)
    h2d = h_flat.reshape(_B, _DIN)

    wc = outw[0, 1:1 + _NPAIRS]
    Wsym = (jnp.zeros((_NF, _NF), jnp.float32)
            .at[_I1, _I2].set(wc).at[_I2, _I1].set(wc))
    Wbig = jnp.kron(Wsym, jnp.eye(_D, dtype=jnp.float32))
    w1mat = l1w.T
    l2wT = l2w.T
    wd = outw[0, 1 + _NPAIRS:]
    m3row = (l3w.T @ wd).reshape(1, 64)
    c0 = outb[0] + jnp.dot(l3b, wd)
    scl = jnp.stack([outw[0, 0], c0]).reshape(1, 2)

    out = _tc_dense(h2d, e1.reshape(_B, 1), w1mat, Wbig, l2wT,
                    bn1g.reshape(1, 128), bn1b.reshape(1, 128),
                    bn2g.reshape(1, 64), bn2b.reshape(1, 64), m3row, scl)
    return out[2].reshape(_B)


# speedup vs baseline: 1.0018x; 1.0018x over previous
import functools

import numpy as np
import jax
import jax.numpy as jnp
from jax import lax
from jax.experimental import pallas as pl
from jax.experimental.pallas import tpu as pltpu
from jax.experimental.pallas import tpu_sc as plsc

_B = 16384
_NF = 26
_D = 16
_DIN = _NF * _D
_NPAIRS = _NF * (_NF - 1) // 2
_V = 2600000

_NC, _NS = 2, 16
_NW = _NC * _NS
_BPW = _B // _NW
_CH = 128
_NCH = _BPW // _CH
_ROWS = _CH * _NF

_CW = 1024
_NCHUNK = _V // _CW
_NITER6 = 84
_TAILC = _V - _NCHUNK * _CW

_bB = 1024
_NB = _B // _bB

_OFFSETS = (100000 * np.arange(_NF, dtype=np.int32))
_I1, _I2 = np.triu_indices(_NF, k=1)


def _tr_body(w2t_h, tail_h, out_h, ab0, ab1, ab2, o0, o1,
             si0, si1, si2, so0, so1):
    wid = lax.axis_index("s") * _NC + lax.axis_index("c")
    abs_ = (ab0, ab1, ab2)
    os_ = (o0, o1)
    sis = (si0, si1, si2)
    sos = (so0, so1)

    def start_in(cid, b):
        @pl.when(cid < _NCHUNK)
        def _():
            c0 = pl.multiple_of(cid * _CW, 128)
            pltpu.async_copy(w2t_h.at[:, pl.ds(c0, _CW)], abs_[b], sis[b])

    def wait_in(cid, b):
        @pl.when(cid < _NCHUNK)
        def _():
            pltpu.make_async_copy(w2t_h.at[:, pl.ds(0, _CW)],
                                  abs_[b], sis[b]).wait()

    def wait_out(cid, b):
        @pl.when(jnp.logical_and(cid >= 0, cid < _NCHUNK))
        def _():
            pltpu.make_async_copy(os_[b], out_h.at[pl.ds(0, _CW * _D)],
                                  sos[b]).wait()

    def scatter(bi, bo):
        ab_v, o_v = abs_[bi], os_[bo]

        def scat(g, carry):
            o = g * 16
            rows = (lax.iota(jnp.int32, 16) + o) * _D
            for d in range(_D):
                v = ab_v[d, pl.ds(o, 16)]
                plsc.store_scatter(o_v, [rows + d], v)
            return carry

        lax.fori_loop(0, _CW // 16, scat, 0)

    def half(cid, bi, bo):
        wait_in(cid, bi)
        start_in(cid + 2 * _NW, (bi + 2) % 3)
        wait_out(cid - 2 * _NW, bo)

        @pl.when(cid < _NCHUNK)
        def _():
            scatter(bi, bo)
            c0 = pl.multiple_of(cid * _CW, 128)
            pltpu.async_copy(os_[bo], out_h.at[pl.ds(c0 * _D, _CW * _D)],
                             sos[bo])

    start_in(wid, 0)
    start_in(wid + _NW, 1)

    def body6(t, carry):
        i0 = 6 * t
        for k in range(6):
            half(wid + (i0 + k) * _NW, k % 3, k % 2)
        return carry

    lax.fori_loop(0, _NITER6 // 6, body6, 0)
    wait_out(wid + (_NITER6 - 2) * _NW, 0)
    wait_out(wid + (_NITER6 - 1) * _NW, 1)

    @pl.when(wid == 0)
    def _():
        pltpu.sync_copy(tail_h, o0.at[pl.ds(0, _TAILC * _D)])
        pltpu.sync_copy(o0.at[pl.ds(0, _TAILC * _D)],
                        out_h.at[pl.ds(_NCHUNK * _CW * _D, _TAILC * _D)])


def _sc_transpose(w2t, tail1d):
    mesh = plsc.VectorSubcoreMesh(core_axis_name="c", subcore_axis_name="s")
    kfn = functools.partial(
        pl.kernel, mesh=mesh,
        compiler_params=pltpu.CompilerParams(use_tc_tiling_on_sc=True,
                                             needs_layout_passes=False),
        out_type=jax.ShapeDtypeStruct((_V * _D,), jnp.float32),
        scratch_types=[
            pltpu.VMEM((_D, _CW), jnp.float32),
            pltpu.VMEM((_D, _CW), jnp.float32),
            pltpu.VMEM((_D, _CW), jnp.float32),
            pltpu.VMEM((_CW * _D,), jnp.float32),
            pltpu.VMEM((_CW * _D,), jnp.float32),
            pltpu.SemaphoreType.DMA,
            pltpu.SemaphoreType.DMA,
            pltpu.SemaphoreType.DMA,
            pltpu.SemaphoreType.DMA,
            pltpu.SemaphoreType.DMA,
        ],
    )(_tr_body)
    return kfn(w2t, tail1d)


def _sc_body(W2_h, idx2_h, W1_h, idx1_h, h_out, e1_out,
             idx_v, rows_v, idx1_v, vals_v, acc_v, sem):
    wid = lax.axis_index("s") * _NC + lax.axis_index("c")
    base = wid * _BPW
    for c in range(_NCH):
        i0 = (base + c * _CH) * _NF
        pltpu.sync_copy(idx2_h.at[pl.ds(i0, _ROWS)], idx_v)
        pltpu.async_copy(W2_h.at[idx_v], rows_v, sem).wait()
        pltpu.sync_copy(rows_v, h_out.at[pl.ds(i0, _ROWS)])
    pltpu.sync_copy(idx1_h.at[wid], idx1_v)
    pltpu.async_copy(W1_h.at[idx1_v], vals_v, sem).wait()

    def red(c2, carry):
        o = c2 * 16
        s = vals_v[pl.ds(o, 16)]
        for f in range(1, _NF):
            s = s + vals_v[pl.ds(f * _BPW + o, 16)]
        acc_v[pl.ds(o, 16)] = s
        return carry

    lax.fori_loop(0, _BPW // 16, red, 0)
    pltpu.sync_copy(acc_v, e1_out.at[pl.ds(base, _BPW)])


def _sc_gather(W2row, idx2, W1v, idx1):
    mesh = plsc.VectorSubcoreMesh(core_axis_name="c", subcore_axis_name="s")
    kfn = functools.partial(
        pl.kernel, mesh=mesh,
        compiler_params=pltpu.CompilerParams(use_tc_tiling_on_sc=False),
        out_type=[jax.ShapeDtypeStruct((_B * _NF, _D), jnp.float32),
                  jax.ShapeDtypeStruct((_B,), jnp.float32)],
        scratch_types=[
            pltpu.VMEM((_ROWS,), jnp.int32),
            pltpu.VMEM((_ROWS, _D), jnp.float32),
            pltpu.VMEM((_NF * _BPW,), jnp.int32),
            pltpu.VMEM((_NF * _BPW,), jnp.float32),
            pltpu.VMEM((_BPW,), jnp.float32),
            pltpu.SemaphoreType.DMA,
        ],
    )(_sc_body)
    return kfn(W2row, idx2, W1v, idx1)


def _tc_body(h_ref, e1_ref, w1_ref, wb_ref, l2_ref, g1_ref, b1_ref,
             g2_ref, b2_ref, m3_ref, scl_ref, out_ref,
             pre1_s, pre2_s, cross_s, st1, st2):
    p = pl.program_id(0)
    j = pl.program_id(1)

    @pl.when(p == 0)
    def _p0():
        h = h_ref[...]
        pre1 = jnp.dot(h, w1_ref[...], preferred_element_type=jnp.float32)
        g = jnp.dot(h, wb_ref[...], preferred_element_type=jnp.float32)
        cross_s[pl.ds(j * _bB, _bB), :] = 0.5 * jnp.sum(h * g, axis=1,
                                                        keepdims=True)
        pre1_s[pl.ds(j * _bB, _bB), :] = pre1

        @pl.when(j == 0)
        def _z1():
            st1[...] = jnp.zeros((2, 128), jnp.float32)

        st1[...] += jnp.concatenate(
            [jnp.sum(pre1, 0, keepdims=True),
             jnp.sum(pre1 * pre1, 0, keepdims=True)], axis=0)

    @pl.when(p == 1)
    def _p1():
        mu = st1[0:1, :] * (1.0 / _B)
        var = st1[1:2, :] * (1.0 / _B) - mu * mu
        sc = g1_ref[...] * lax.rsqrt(var + 1e-5)
        pre1 = pre1_s[pl.ds(j * _bB, _bB), :]
        h1 = jnp.maximum((pre1 - mu) * sc + b1_ref[...], 0.0)
        pre2 = jnp.dot(h1, l2_ref[...], preferred_element_type=jnp.float32)
        pre2_s[pl.ds(j * _bB, _bB), :] = pre2

        @pl.when(j == 0)
        def _z2():
            st2[...] = jnp.zeros((2, 64), jnp.float32)

        st2[...] += jnp.concatenate(
            [jnp.sum(pre2, 0, keepdims=True),
             jnp.sum(pre2 * pre2, 0, keepdims=True)], axis=0)

    @pl.when(p == 2)
    def _p2():
        mu2 = st2[0:1, :] * (1.0 / _B)
        var2 = st2[1:2, :] * (1.0 / _B) - mu2 * mu2
        sc2 = g2_ref[...] * lax.rsqrt(var2 + 1e-5)
        pre2 = pre2_s[pl.ds(j * _bB, _bB), :]
        h2 = jnp.maximum((pre2 - mu2) * sc2 + b2_ref[...], 0.0)
        deep = jnp.sum(h2 * m3_ref[...], axis=1, keepdims=True)
        logit = (deep + e1_ref[...] * scl_ref[0:1, 0:1]
                 + cross_s[pl.ds(j * _bB, _bB), :] + scl_ref[0:1, 1:2])
        out_ref[...] = (1.0 / (1.0 + jnp.exp(-logit)))[None]


def _tc_dense(h2d, e1c, w1mat, Wbig, l2wT, g1, b1, g2, b2, m3row, scl):
    z2 = lambda p, j: (0, 0)
    return pl.pallas_call(
        _tc_body,
        grid=(3, _NB),
        in_specs=[
            pl.BlockSpec((_bB, _DIN), lambda p, j: (jnp.where(p == 0, j, 0), 0)),
            pl.BlockSpec((_bB, 1), lambda p, j: (jnp.where(p == 2, j, 0), 0)),
            pl.BlockSpec((_DIN, 128), z2),
            pl.BlockSpec((_DIN, _DIN), z2),
            pl.BlockSpec((128, 64), z2),
            pl.BlockSpec((1, 128), z2),
            pl.BlockSpec((1, 128), z2),
            pl.BlockSpec((1, 64), z2),
            pl.BlockSpec((1, 64), z2),
            pl.BlockSpec((1, 64), z2),
            pl.BlockSpec((1, 2), z2),
        ],
        out_specs=pl.BlockSpec((1, _bB, 1), lambda p, j: (p, j, 0)),
        out_shape=jax.ShapeDtypeStruct((3, _B, 1), jnp.float32),
        scratch_shapes=[
            pltpu.VMEM((_B, 128), jnp.float32),
            pltpu.VMEM((_B, 64), jnp.float32),
            pltpu.VMEM((_B, 1), jnp.float32),
            pltpu.VMEM((2, 128), jnp.float32),
            pltpu.VMEM((2, 64), jnp.float32),
        ],
    )(h2d, e1c, w1mat, Wbig, l2wT, g1, b1, g2, b2, m3row, scl)


def kernel(x, W1, W2, l1w, l1b, bn1g, bn1b, l2w, l2b, bn2g, bn2b,
           l3w, l3b, outw, outb):
    offs = jnp.asarray(_OFFSETS)
    xo = x + offs[None, :]
    idx2 = xo.reshape(_B * _NF)
    idx1 = xo.reshape(_NW, _BPW, _NF).transpose(0, 2, 1).reshape(_NW, _NF * _BPW)
    W1v = W1.reshape(-1)

    tail1d = W2[_NCHUNK * _CW:, :].reshape(_TAILC * _D)
    W2row = _sc_transpose(W2.T, tail1d).reshape(_V, _D)

    h_flat, e1 = _sc_gather(W2row, idx2, W1v, idx1)
    h2d = h_flat.reshape(_B, _DIN)

    wc = outw[0, 1:1 + _NPAIRS]
    Wsym = (jnp.zeros((_NF, _NF), jnp.float32)
            .at[_I1, _I2].set(wc).at[_I2, _I1].set(wc))
    Wbig = jnp.kron(Wsym, jnp.eye(_D, dtype=jnp.float32))
    w1mat = l1w.T
    l2wT = l2w.T
    wd = outw[0, 1 + _NPAIRS:]
    m3row = (l3w.T @ wd).reshape(1, 64)
    c0 = outb[0] + jnp.dot(l3b, wd)
    scl = jnp.stack([outw[0, 0], c0]).reshape(1, 2)

    out = _tc_dense(h2d, e1.reshape(_B, 1), w1mat, Wbig, l2wT,
                    bn1g.reshape(1, 128), bn1b.reshape(1, 128),
                    bn2g.reshape(1, 64), bn2b.reshape(1, 64), m3row, scl)
    return out[2].reshape(_B)


# speedup vs baseline: 1.0057x; 1.0039x over previous
import functools

import numpy as np
import jax
import jax.numpy as jnp
from jax import lax
from jax.experimental import pallas as pl
from jax.experimental.pallas import tpu as pltpu
from jax.experimental.pallas import tpu_sc as plsc

_B = 16384
_NF = 26
_D = 16
_DIN = _NF * _D
_NPAIRS = _NF * (_NF - 1) // 2
_V = 2600000

_NC, _NS = 2, 16
_NW = _NC * _NS
_BPW = _B // _NW
_CH = 128
_NCH = _BPW // _CH
_ROWS = _CH * _NF

_CW = 1024
_NCHUNK = _V // _CW
_NITER6 = 84
_TAILC = _V - _NCHUNK * _CW

_bB = 1024
_NB = _B // _bB

_OFFSETS = (100000 * np.arange(_NF, dtype=np.int32))
_I1, _I2 = np.triu_indices(_NF, k=1)


def _tr_body(w2t_h, tail_h, out_h, ab0, ab1, ab2, o0, o1,
             si0, si1, si2, so0, so1):
    wid = lax.axis_index("s") * _NC + lax.axis_index("c")
    abs_ = (ab0, ab1, ab2)
    os_ = (o0, o1)
    sis = (si0, si1, si2)
    sos = (so0, so1)

    def start_in(cid, b):
        @pl.when(cid < _NCHUNK)
        def _():
            c0 = pl.multiple_of(cid * _CW, 128)
            pltpu.async_copy(w2t_h.at[:, pl.ds(c0, _CW)], abs_[b], sis[b])

    def wait_in(cid, b):
        @pl.when(cid < _NCHUNK)
        def _():
            pltpu.make_async_copy(w2t_h.at[:, pl.ds(0, _CW)],
                                  abs_[b], sis[b]).wait()

    def wait_out(cid, b):
        @pl.when(jnp.logical_and(cid >= 0, cid < _NCHUNK))
        def _():
            pltpu.make_async_copy(os_[b], out_h.at[pl.ds(0, _CW * _D)],
                                  sos[b]).wait()

    def scatter(bi, bo):
        ab_v, o_v = abs_[bi], os_[bo]

        def scat(g, carry):
            o = g * 16
            rows = (lax.iota(jnp.int32, 16) + o) * _D
            for d in range(_D):
                v = ab_v[d, pl.ds(o, 16)]
                plsc.store_scatter(o_v, [rows + d], v)
            return carry

        lax.fori_loop(0, _CW // 16, scat, 0)

    def half(cid, b):
        wait_in(cid, b)
        start_in(cid + _NW, 1 - b)
        wait_out(cid - 2 * _NW, b)

        @pl.when(cid < _NCHUNK)
        def _():
            scatter(b, b)
            c0 = pl.multiple_of(cid * _CW, 128)
            pltpu.async_copy(os_[b], out_h.at[pl.ds(c0 * _D, _CW * _D)],
                             sos[b])

    start_in(wid, 0)

    def body2(t, carry):
        i0 = 2 * t
        half(wid + i0 * _NW, 0)
        half(wid + (i0 + 1) * _NW, 1)
        return carry

    lax.fori_loop(0, _NITER6 // 2, body2, 0)
    wait_out(wid + (_NITER6 - 2) * _NW, 0)
    wait_out(wid + (_NITER6 - 1) * _NW, 1)

    @pl.when(wid == 0)
    def _():
        pltpu.sync_copy(tail_h, o0.at[pl.ds(0, _TAILC * _D)])
        pltpu.sync_copy(o0.at[pl.ds(0, _TAILC * _D)],
                        out_h.at[pl.ds(_NCHUNK * _CW * _D, _TAILC * _D)])


def _sc_transpose(w2t, tail1d):
    mesh = plsc.VectorSubcoreMesh(core_axis_name="c", subcore_axis_name="s")
    kfn = functools.partial(
        pl.kernel, mesh=mesh,
        compiler_params=pltpu.CompilerParams(use_tc_tiling_on_sc=True,
                                             needs_layout_passes=False),
        out_type=jax.ShapeDtypeStruct((_V * _D,), jnp.float32),
        scratch_types=[
            pltpu.VMEM((_D, _CW), jnp.float32),
            pltpu.VMEM((_D, _CW), jnp.float32),
            pltpu.VMEM((_D, _CW), jnp.float32),
            pltpu.VMEM((_CW * _D,), jnp.float32),
            pltpu.VMEM((_CW * _D,), jnp.float32),
            pltpu.SemaphoreType.DMA,
            pltpu.SemaphoreType.DMA,
            pltpu.SemaphoreType.DMA,
            pltpu.SemaphoreType.DMA,
            pltpu.SemaphoreType.DMA,
        ],
    )(_tr_body)
    return kfn(w2t, tail1d)


def _sc_body(W2_h, idx2_h, W1_h, idx1_h, h_out, e1_out,
             idx_v, rows_v, idx1_v, vals_v, acc_v, sem):
    wid = lax.axis_index("s") * _NC + lax.axis_index("c")
    base = wid * _BPW
    for c in range(_NCH):
        i0 = (base + c * _CH) * _NF
        pltpu.sync_copy(idx2_h.at[pl.ds(i0, _ROWS)], idx_v)
        pltpu.async_copy(W2_h.at[idx_v], rows_v, sem).wait()
        pltpu.sync_copy(rows_v, h_out.at[pl.ds(i0, _ROWS)])
    pltpu.sync_copy(idx1_h.at[wid], idx1_v)
    pltpu.async_copy(W1_h.at[idx1_v], vals_v, sem).wait()

    def red(c2, carry):
        o = c2 * 16
        s = vals_v[pl.ds(o, 16)]
        for f in range(1, _NF):
            s = s + vals_v[pl.ds(f * _BPW + o, 16)]
        acc_v[pl.ds(o, 16)] = s
        return carry

    lax.fori_loop(0, _BPW // 16, red, 0)
    pltpu.sync_copy(acc_v, e1_out.at[pl.ds(base, _BPW)])


def _sc_gather(W2row, idx2, W1v, idx1):
    mesh = plsc.VectorSubcoreMesh(core_axis_name="c", subcore_axis_name="s")
    kfn = functools.partial(
        pl.kernel, mesh=mesh,
        compiler_params=pltpu.CompilerParams(use_tc_tiling_on_sc=False),
        out_type=[jax.ShapeDtypeStruct((_B * _NF, _D), jnp.float32),
                  jax.ShapeDtypeStruct((_B,), jnp.float32)],
        scratch_types=[
            pltpu.VMEM((_ROWS,), jnp.int32),
            pltpu.VMEM((_ROWS, _D), jnp.float32),
            pltpu.VMEM((_NF * _BPW,), jnp.int32),
            pltpu.VMEM((_NF * _BPW,), jnp.float32),
            pltpu.VMEM((_BPW,), jnp.float32),
            pltpu.SemaphoreType.DMA,
        ],
    )(_sc_body)
    return kfn(W2row, idx2, W1v, idx1)


def _tc_body(h_ref, e1_ref, w1_ref, wb_ref, l2_ref, g1_ref, b1_ref,
             g2_ref, b2_ref, m3_ref, scl_ref, out_ref,
             pre1_s, pre2_s, cross_s, st1, st2):
    p = pl.program_id(0)
    j = pl.program_id(1)

    @pl.when(p == 0)
    def _p0():
        h = h_ref[...]
        pre1 = jnp.dot(h, w1_ref[...], preferred_element_type=jnp.float32)
        g = jnp.dot(h, wb_ref[...], preferred_element_type=jnp.float32)
        cross_s[pl.ds(j * _bB, _bB), :] = 0.5 * jnp.sum(h * g, axis=1,
                                                        keepdims=True)
        pre1_s[pl.ds(j * _bB, _bB), :] = pre1

        @pl.when(j == 0)
        def _z1():
            st1[...] = jnp.zeros((2, 128), jnp.float32)

        st1[...] += jnp.concatenate(
            [jnp.sum(pre1, 0, keepdims=True),
             jnp.sum(pre1 * pre1, 0, keepdims=True)], axis=0)

    @pl.when(p == 1)
    def _p1():
        mu = st1[0:1, :] * (1.0 / _B)
        var = st1[1:2, :] * (1.0 / _B) - mu * mu
        sc = g1_ref[...] * lax.rsqrt(var + 1e-5)
        pre1 = pre1_s[pl.ds(j * _bB, _bB), :]
        h1 = jnp.maximum((pre1 - mu) * sc + b1_ref[...], 0.0)
        pre2 = jnp.dot(h1, l2_ref[...], preferred_element_type=jnp.float32)
        pre2_s[pl.ds(j * _bB, _bB), :] = pre2

        @pl.when(j == 0)
        def _z2():
            st2[...] = jnp.zeros((2, 64), jnp.float32)

        st2[...] += jnp.concatenate(
            [jnp.sum(pre2, 0, keepdims=True),
             jnp.sum(pre2 * pre2, 0, keepdims=True)], axis=0)

    @pl.when(p == 2)
    def _p2():
        mu2 = st2[0:1, :] * (1.0 / _B)
        var2 = st2[1:2, :] * (1.0 / _B) - mu2 * mu2
        sc2 = g2_ref[...] * lax.rsqrt(var2 + 1e-5)
        pre2 = pre2_s[pl.ds(j * _bB, _bB), :]
        h2 = jnp.maximum((pre2 - mu2) * sc2 + b2_ref[...], 0.0)
        deep = jnp.sum(h2 * m3_ref[...], axis=1, keepdims=True)
        logit = (deep + e1_ref[...] * scl_ref[0:1, 0:1]
                 + cross_s[pl.ds(j * _bB, _bB), :] + scl_ref[0:1, 1:2])
        out_ref[...] = (1.0 / (1.0 + jnp.exp(-logit)))[None]


def _tc_dense(h2d, e1c, w1mat, Wbig, l2wT, g1, b1, g2, b2, m3row, scl):
    z2 = lambda p, j: (0, 0)
    return pl.pallas_call(
        _tc_body,
        grid=(3, _NB),
        in_specs=[
            pl.BlockSpec((_bB, _DIN), lambda p, j: (jnp.where(p == 0, j, 0), 0)),
            pl.BlockSpec((_bB, 1), lambda p, j: (jnp.where(p == 2, j, 0), 0)),
            pl.BlockSpec((_DIN, 128), z2),
            pl.BlockSpec((_DIN, _DIN), z2),
            pl.BlockSpec((128, 64), z2),
            pl.BlockSpec((1, 128), z2),
            pl.BlockSpec((1, 128), z2),
            pl.BlockSpec((1, 64), z2),
            pl.BlockSpec((1, 64), z2),
            pl.BlockSpec((1, 64), z2),
            pl.BlockSpec((1, 2), z2),
        ],
        out_specs=pl.BlockSpec((1, _bB, 1), lambda p, j: (p, j, 0)),
        out_shape=jax.ShapeDtypeStruct((3, _B, 1), jnp.float32),
        scratch_shapes=[
            pltpu.VMEM((_B, 128), jnp.float32),
            pltpu.VMEM((_B, 64), jnp.float32),
            pltpu.VMEM((_B, 1), jnp.float32),
            pltpu.VMEM((2, 128), jnp.float32),
            pltpu.VMEM((2, 64), jnp.float32),
        ],
    )(h2d, e1c, w1mat, Wbig, l2wT, g1, b1, g2, b2, m3row, scl)


def kernel(x, W1, W2, l1w, l1b, bn1g, bn1b, l2w, l2b, bn2g, bn2b,
           l3w, l3b, outw, outb):
    offs = jnp.asarray(_OFFSETS)
    xo = x + offs[None, :]
    idx2 = xo.reshape(_B * _NF)
    idx1 = xo.reshape(_NW, _BPW, _NF).transpose(0, 2, 1).reshape(_NW, _NF * _BPW)
    W1v = W1.reshape(-1)

    tail1d = W2[_NCHUNK * _CW:, :].reshape(_TAILC * _D)
    W2row = _sc_transpose(W2.T, tail1d).reshape(_V, _D)

    h_flat, e1 = _sc_gather(W2row, idx2, W1v, idx1)
    h2d = h_flat.reshape(_B, _DIN)

    wc = outw[0, 1:1 + _NPAIRS]
    Wsym = (jnp.zeros((_NF, _NF), jnp.float32)
            .at[_I1, _I2].set(wc).at[_I2, _I1].set(wc))
    Wbig = jnp.kron(Wsym, jnp.eye(_D, dtype=jnp.float32))
    w1mat = l1w.T
    l2wT = l2w.T
    wd = outw[0, 1 + _NPAIRS:]
    m3row = (l3w.T @ wd).reshape(1, 64)
    c0 = outb[0] + jnp.dot(l3b, wd)
    scl = jnp.stack([outw[0, 0], c0]).reshape(1, 2)

    out = _tc_dense(h2d, e1.reshape(_B, 1), w1mat, Wbig, l2wT,
                    bn1g.reshape(1, 128), bn1b.reshape(1, 128),
                    bn2g.reshape(1, 64), bn2b.reshape(1, 64), m3row, scl)
    return out[2].reshape(_B)
